# baseline (speedup 1.0000x reference)
import jax
import jax.numpy as jnp
from jax.experimental import pallas as pl

_B, _C_IN, _C_OUT, _H, _W = 8, 16, 16, 512, 512
_TILE = 16384


def _spconv_kern(x_ref, w_ref, b_ref, o_ref):
    xb = x_ref[0]
    w = w_ref[...]
    out = jax.lax.dot_general(
        w, xb, (((1,), (0,)), ((), ())), preferred_element_type=jnp.float32
    )
    out = out + b_ref[...]
    mask = jnp.any(xb != 0, axis=0, keepdims=True)
    o_ref[0] = jnp.where(mask, out, 0.0)


def kernel(x, W, b):
    HW = _H * _W
    xr = x.reshape(_B, _C_IN, HW)
    b2 = b.reshape(_C_OUT, 1)
    out = pl.pallas_call(
        _spconv_kern,
        grid=(_B, HW // _TILE),
        in_specs=[
            pl.BlockSpec((1, _C_IN, _TILE), lambda i, j: (i, 0, j)),
            pl.BlockSpec((_C_OUT, _C_IN), lambda i, j: (0, 0)),
            pl.BlockSpec((_C_OUT, 1), lambda i, j: (0, 0)),
        ],
        out_specs=pl.BlockSpec((1, _C_OUT, _TILE), lambda i, j: (i, 0, j)),
        out_shape=jax.ShapeDtypeStruct((_B, _C_OUT, HW), jnp.float32),
    )(xr, W, b2)
    return out.reshape(_B, _C_OUT, _H, _W)


# speedup vs baseline: 1.2294x; 1.2294x over previous
import jax
import jax.numpy as jnp
from jax.experimental import pallas as pl

_B, _C_IN, _C_OUT, _H, _W = 8, 16, 16, 512, 512
_TILE = 32768


def _spconv_kern(x_ref, wcat_ref, o_ref):
    xb = x_ref[0]
    ones8 = jnp.ones((8, _C_IN), dtype=jnp.bfloat16)
    a = jnp.abs(xb).astype(jnp.bfloat16)
    s = jax.lax.dot_general(
        ones8, a, (((1,), (0,)), ((), ())), preferred_element_type=jnp.float32
    )
    maskf = jnp.where(s > 0, 1.0, 0.0).astype(jnp.float32)
    aug = jnp.concatenate([xb, maskf], axis=0)
    out = jax.lax.dot_general(
        wcat_ref[...], aug, (((1,), (0,)), ((), ())),
        preferred_element_type=jnp.float32,
    )
    o_ref[0] = out


def kernel(x, W, b):
    HW = _H * _W
    xr = x.reshape(_B, _C_IN, HW)
    wcat = jnp.concatenate(
        [W, b.reshape(_C_OUT, 1), jnp.zeros((_C_OUT, 7), jnp.float32)], axis=1
    )
    out = pl.pallas_call(
        _spconv_kern,
        grid=(_B, HW // _TILE),
        in_specs=[
            pl.BlockSpec((1, _C_IN, _TILE), lambda i, j: (i, 0, j)),
            pl.BlockSpec((_C_OUT, _C_IN + 8), lambda i, j: (0, 0)),
        ],
        out_specs=pl.BlockSpec((1, _C_OUT, _TILE), lambda i, j: (i, 0, j)),
        out_shape=jax.ShapeDtypeStruct((_B, _C_OUT, HW), jnp.float32),
    )(xr, wcat)
    return out.reshape(_B, _C_OUT, _H, _W)


# speedup vs baseline: 3.8260x; 3.1121x over previous
import jax
import jax.numpy as jnp
from jax.experimental import pallas as pl

_B, _C_IN, _C_OUT, _H, _W = 8, 16, 16, 512, 512
_TH = 64


def _spconv_kern(x_ref, wcat_ref, o_ref):
    xb = x_ref[0].reshape(_C_IN, _TH * _W)
    ones8 = jnp.ones((8, _C_IN), dtype=jnp.bfloat16)
    a = jnp.abs(xb).astype(jnp.bfloat16)
    s = jax.lax.dot_general(
        ones8, a, (((1,), (0,)), ((), ())), preferred_element_type=jnp.float32
    )
    maskf = jnp.where(s > 0, 1.0, 0.0).astype(jnp.float32)
    aug = jnp.concatenate([xb, maskf], axis=0)
    out = jax.lax.dot_general(
        wcat_ref[...], aug, (((1,), (0,)), ((), ())),
        preferred_element_type=jnp.float32,
    )
    o_ref[0] = out.reshape(_C_OUT, _TH, _W)


def kernel(x, W, b):
    wcat = jnp.concatenate(
        [W, b.reshape(_C_OUT, 1), jnp.zeros((_C_OUT, 7), jnp.float32)], axis=1
    )
    out = pl.pallas_call(
        _spconv_kern,
        grid=(_B, _H // _TH),
        in_specs=[
            pl.BlockSpec((1, _C_IN, _TH, _W), lambda i, j: (i, 0, j, 0)),
            pl.BlockSpec((_C_OUT, _C_IN + 8), lambda i, j: (0, 0)),
        ],
        out_specs=pl.BlockSpec((1, _C_OUT, _TH, _W), lambda i, j: (i, 0, j, 0)),
        out_shape=jax.ShapeDtypeStruct((_B, _C_OUT, _H, _W), jnp.float32),
    )(x, wcat)
    return out


# speedup vs baseline: 4.4042x; 1.1511x over previous
import jax
import jax.numpy as jnp
from jax.experimental import pallas as pl

_B, _C_IN, _C_OUT, _H, _W = 8, 16, 16, 512, 512
_TH = 128


def _spconv_kern(x_ref, wcat_ref, o_ref):
    xb = x_ref[0].reshape(_C_IN, _TH * _W)
    ones8 = jnp.ones((8, _C_IN), dtype=jnp.bfloat16)
    a = jnp.abs(xb).astype(jnp.bfloat16)
    s = jax.lax.dot_general(
        ones8, a, (((1,), (0,)), ((), ())), preferred_element_type=jnp.float32
    )
    maskf = jnp.where(s > 0, 1.0, 0.0).astype(jnp.float32)
    aug = jnp.concatenate([xb, maskf], axis=0)
    out = jax.lax.dot_general(
        wcat_ref[...], aug, (((1,), (0,)), ((), ())),
        preferred_element_type=jnp.float32,
    )
    o_ref[0] = out.reshape(_C_OUT, _TH, _W)


def kernel(x, W, b):
    wcat = jnp.concatenate(
        [W, b.reshape(_C_OUT, 1), jnp.zeros((_C_OUT, 7), jnp.float32)], axis=1
    )
    out = pl.pallas_call(
        _spconv_kern,
        grid=(_B, _H // _TH),
        in_specs=[
            pl.BlockSpec((1, _C_IN, _TH, _W), lambda i, j: (i, 0, j, 0)),
            pl.BlockSpec((_C_OUT, _C_IN + 8), lambda i, j: (0, 0)),
        ],
        out_specs=pl.BlockSpec((1, _C_OUT, _TH, _W), lambda i, j: (i, 0, j, 0)),
        out_shape=jax.ShapeDtypeStruct((_B, _C_OUT, _H, _W), jnp.float32),
    )(x, wcat)
    return out


# speedup vs baseline: 4.7357x; 1.0753x over previous
import jax
import jax.numpy as jnp
from jax.experimental import pallas as pl

_B, _C_IN, _C_OUT, _H, _W = 8, 16, 16, 512, 512
_TH = 256


def _spconv_kern(x_ref, wcat_ref, o_ref):
    xb = x_ref[0].reshape(_C_IN, _TH * _W)
    ones8 = jnp.ones((8, _C_IN), dtype=jnp.bfloat16)
    a = jnp.abs(xb).astype(jnp.bfloat16)
    s = jax.lax.dot_general(
        ones8, a, (((1,), (0,)), ((), ())), preferred_element_type=jnp.float32
    )
    maskf = jnp.where(s > 0, 1.0, 0.0).astype(jnp.float32)
    aug = jnp.concatenate([xb, maskf], axis=0)
    out = jax.lax.dot_general(
        wcat_ref[...], aug, (((1,), (0,)), ((), ())),
        preferred_element_type=jnp.float32,
    )
    o_ref[0] = out.reshape(_C_OUT, _TH, _W)


def kernel(x, W, b):
    wcat = jnp.concatenate(
        [W, b.reshape(_C_OUT, 1), jnp.zeros((_C_OUT, 7), jnp.float32)], axis=1
    )
    out = pl.pallas_call(
        _spconv_kern,
        grid=(_B, _H // _TH),
        in_specs=[
            pl.BlockSpec((1, _C_IN, _TH, _W), lambda i, j: (i, 0, j, 0)),
            pl.BlockSpec((_C_OUT, _C_IN + 8), lambda i, j: (0, 0)),
        ],
        out_specs=pl.BlockSpec((1, _C_OUT, _TH, _W), lambda i, j: (i, 0, j, 0)),
        out_shape=jax.ShapeDtypeStruct((_B, _C_OUT, _H, _W), jnp.float32),
    )(x, wcat)
    return out
